# baseline (speedup 1.0000x reference)
import functools

import jax
import jax.numpy as jnp
from jax import lax
from jax.experimental import pallas as pl
from jax.experimental.pallas import tpu as pltpu
from jax.experimental.pallas import tpu_sc as plsc

N = 10000
E = 320000
D_IN = 128
D_HID = 128
D_OUT = 64

N_PAD = 10240
NC = 2
NS = 16
NW = NC * NS
C = 128
NCH = 79
CH_TOT = NCH * NW
E_PAD = CH_TOT * C
CH_F = 106
CH_S = CH_TOT // NS - CH_F
PH = 40
RPT = N_PAD // NS

_MESH = plsc.VectorSubcoreMesh(core_axis_name="c", subcore_axis_name="s")


def _fill(ref, rows, val):
    def body(k, _):
        ref[k // 8, pl.ds((k % 8) * 16, 16)] = jnp.full((16,), val, jnp.float32)
        return 0

    lax.fori_loop(0, rows * 8, body, 0)


def _zero_acc(zbuf_v, acc_sh, sid):
    _fill(zbuf_v, C, 0.0)

    def zcp(k, _):
        pltpu.sync_copy(zbuf_v, acc_sh.at[pl.ds(sid * RPT + k * C, C)])
        return 0

    lax.fori_loop(0, RPT // C, zcp, 0)
    plsc.subcore_barrier()


def _copy_out(acc_sh, out_hbm, cid, sid):
    plsc.subcore_barrier()
    pltpu.sync_copy(
        acc_sh.at[pl.ds(sid * RPT, RPT)],
        out_hbm.at[cid, pl.ds(sid * RPT, RPT)],
    )


@functools.partial(
    pl.kernel,
    mesh=_MESH,
    out_type=jax.ShapeDtypeStruct((NC, N_PAD, D_HID), jnp.float32),
    scratch_types=[
        pltpu.VMEM((PH, 2, C), jnp.int32),
        pltpu.VMEM((C, D_HID), jnp.float32),
        pltpu.VMEM_SHARED((N_PAD, D_HID), jnp.float32),
        pltpu.SemaphoreType.DMA,
        pltpu.SemaphoreType.DMA,
    ],
)
def _deg_sc(ei_hbm, out_hbm, idx_v, ones_v, acc_sh, sema, semb):
    cid = lax.axis_index("c")
    sid = lax.axis_index("s")
    wid = sid * NC + cid
    _zero_acc(ones_v, acc_sh, sid)
    _fill(ones_v, C, 1.0)

    def scat(ch, sem):
        pltpu.async_copy(ones_v, acc_sh.at[idx_v.at[ch, 1]], sem, add=True)

    def drain(sem):
        pltpu.make_async_copy(ones_v, acc_sh.at[idx_v.at[0, 1]], sem).wait()

    for off in range(0, NCH, PH):
        n = min(PH, NCH - off)
        pltpu.sync_copy(ei_hbm.at[wid, pl.ds(off, n)],
                        idx_v.at[pl.ds(0, n)])

        def pair(j, _):
            scat(2 * j, sema)
            scat(2 * j + 1, semb)
            drain(sema)
            drain(semb)
            return 0

        lax.fori_loop(0, n // 2, pair, 0)
        if n % 2:
            scat(n - 1, sema)
            drain(sema)
    _copy_out(acc_sh, out_hbm, cid, sid)


@functools.partial(
    pl.kernel,
    mesh=_MESH,
    out_type=jax.ShapeDtypeStruct((NC, N_PAD, D_HID), jnp.float32),
    scratch_types=[
        pltpu.VMEM((PH, 2, C), jnp.int32),
        pltpu.VMEM((C, D_HID), jnp.float32),
        pltpu.VMEM((C, D_HID), jnp.float32),
        pltpu.VMEM_SHARED((N_PAD, D_HID), jnp.float32),
        pltpu.SemaphoreType.DMA,
        pltpu.SemaphoreType.DMA,
    ],
)
def _scatter_sc(g_hbm, eif_hbm, eis_hbm, out_hbm, idx_v, rows_a, rows_b,
                acc_sh, sema, semb):
    cid = lax.axis_index("c")
    sid = lax.axis_index("s")
    _zero_acc(rows_a, acc_sh, sid)

    def gat(ch, rows, sem):
        pltpu.async_copy(g_hbm.at[idx_v.at[ch, 0]], rows, sem)

    def wt(rows, sem):
        pltpu.make_async_copy(g_hbm.at[idx_v.at[0, 0]], rows, sem).wait()

    def scat(ch, rows):
        pltpu.sync_copy(rows, acc_sh.at[idx_v.at[ch, 1]], add=True)

    def run(ei_hbm, total):
        for off in range(0, total, PH):
            n = min(PH, total - off)
            pltpu.sync_copy(ei_hbm.at[sid, pl.ds(off, n)],
                            idx_v.at[pl.ds(0, n)])
            gat(0, rows_a, sema)

            def pair(j, _):
                a = 2 * j
                wt(rows_a, sema)
                gat(a + 1, rows_b, semb)
                scat(a, rows_a)
                wt(rows_b, semb)
                gat(a + 2, rows_a, sema)
                scat(a + 1, rows_b)
                return 0

            npair = (n - 1) // 2
            lax.fori_loop(0, npair, pair, 0)
            if n % 2:
                wt(rows_a, sema)
                scat(n - 1, rows_a)
            else:
                wt(rows_a, sema)
                gat(n - 1, rows_b, semb)
                scat(n - 2, rows_a)
                wt(rows_b, semb)
                scat(n - 1, rows_b)

    @pl.when(cid == 0)
    def _():
        run(eif_hbm, CH_F)

    @pl.when(cid == 1)
    def _():
        run(eis_hbm, CH_S)

    _copy_out(acc_sh, out_hbm, cid, sid)


_BR = 1024
_G = N_PAD // _BR


def _rs(deg0, deg1):
    return lax.rsqrt(deg0[:, 0:1] + deg1[:, 0:1] + 1.0)


def _z1_body(deg0_ref, deg1_ref, x_ref, w_ref, z_ref):
    rs = _rs(deg0_ref[...], deg1_ref[...])
    z_ref[...] = jnp.dot(x_ref[...], w_ref[...],
                         preferred_element_type=jnp.float32) * rs


def _mid_body(deg0_ref, deg1_ref, s0_ref, s1_ref, z_ref, b_ref, w_ref, o_ref):
    rs = _rs(deg0_ref[...], deg1_ref[...])
    h = jax.nn.relu(rs * (s0_ref[...] + s1_ref[...] + z_ref[...]) + b_ref[...])
    o_ref[...] = jnp.dot(h, w_ref[...], preferred_element_type=jnp.float32) * rs


def _fin_body(deg0_ref, deg1_ref, s0_ref, s1_ref, z_ref, b_ref, w_ref, bf_ref,
              o_ref):
    rs = _rs(deg0_ref[...], deg1_ref[...])
    h = jax.nn.relu(rs * (s0_ref[...] + s1_ref[...] + z_ref[...]) + b_ref[...])
    o_ref[...] = jnp.dot(h, w_ref[...],
                         preferred_element_type=jnp.float32) + bf_ref[...]


def _row_spec(w):
    return pl.BlockSpec((_BR, w), lambda i: (i, 0))


def _full_spec(r, c):
    return pl.BlockSpec((r, c), lambda i: (0, 0))


_z1_call = pl.pallas_call(
    _z1_body,
    grid=(_G,),
    in_specs=[_row_spec(D_HID), _row_spec(D_HID), _row_spec(D_IN),
              _full_spec(D_IN, D_HID)],
    out_specs=_row_spec(D_HID),
    out_shape=jax.ShapeDtypeStruct((N_PAD, D_HID), jnp.float32),
)

_mid_call = pl.pallas_call(
    _mid_body,
    grid=(_G,),
    in_specs=[_row_spec(D_HID), _row_spec(D_HID), _row_spec(D_HID),
              _row_spec(D_HID), _row_spec(D_HID), _full_spec(1, D_HID),
              _full_spec(D_HID, D_HID)],
    out_specs=_row_spec(D_HID),
    out_shape=jax.ShapeDtypeStruct((N_PAD, D_HID), jnp.float32),
)

_fin_call = pl.pallas_call(
    _fin_body,
    grid=(_G,),
    in_specs=[_row_spec(D_HID), _row_spec(D_HID), _row_spec(D_HID),
              _row_spec(D_HID), _row_spec(D_HID), _full_spec(1, D_HID),
              _full_spec(D_HID, D_OUT), _full_spec(1, D_OUT)],
    out_specs=_row_spec(D_OUT),
    out_shape=jax.ShapeDtypeStruct((N_PAD, D_OUT), jnp.float32),
)


def kernel(x, edge_index, W1, b1, W2, b2, Wf, bf):
    src = edge_index[0].astype(jnp.int32)
    dst = edge_index[1].astype(jnp.int32)
    src_p = jnp.concatenate(
        [src, jnp.zeros((E_PAD - E,), jnp.int32)]).reshape(CH_TOT, 1, C)
    junk = N + jnp.arange(E_PAD - E, dtype=jnp.int32) % (N_PAD - N)
    dst_p = jnp.concatenate([dst, junk]).reshape(CH_TOT, 1, C)
    ei = jnp.concatenate([src_p, dst_p], axis=2)
    ei4 = ei.reshape(NW, NCH, 2, C)
    ei_f = ei[:NS * CH_F].reshape(NS, CH_F, 2, C)
    ei_s = ei[NS * CH_F:].reshape(NS, CH_S, 2, C)
    x_p = jnp.pad(x, ((0, N_PAD - N), (0, 0)))

    deg_p = _deg_sc(ei4)
    deg0, deg1 = deg_p[0], deg_p[1]

    z1 = _z1_call(deg0, deg1, x_p, W1)
    s = _scatter_sc(z1, ei_f, ei_s)
    z2 = _mid_call(deg0, deg1, s[0], s[1], z1, b1.reshape(1, -1), W2)
    s2 = _scatter_sc(z2, ei_f, ei_s)
    out = _fin_call(deg0, deg1, s2[0], s2[1], z2, b2.reshape(1, -1), Wf,
                    bf.reshape(1, -1))
    return out[:N]


# speedup vs baseline: 1.6445x; 1.6445x over previous
import functools

import jax
import jax.numpy as jnp
from jax import lax
from jax.experimental import pallas as pl
from jax.experimental.pallas import tpu as pltpu
from jax.experimental.pallas import tpu_sc as plsc

N = 10000
E = 320000
D_IN = 128
D_HID = 128
D_OUT = 64

N_PAD = 10240
NC = 2
NS = 16
NW = NC * NS
C = 128
NCH = 79
CH_TOT = NCH * NW
E_PAD = CH_TOT * C
CH_F = 79
CH_S = CH_TOT // NS - CH_F
PH = 40
RPT = N_PAD // NS

_MESH = plsc.VectorSubcoreMesh(core_axis_name="c", subcore_axis_name="s")


def _fill(ref, rows, val):
    def body(k, _):
        ref[k // 8, pl.ds((k % 8) * 16, 16)] = jnp.full((16,), val, jnp.float32)
        return 0

    lax.fori_loop(0, rows * 8, body, 0)


def _zero_acc(zbuf_v, acc_sh, sid):
    _fill(zbuf_v, C, 0.0)

    def zcp(k, _):
        pltpu.sync_copy(zbuf_v, acc_sh.at[pl.ds(sid * RPT + k * C, C)])
        return 0

    lax.fori_loop(0, RPT // C, zcp, 0)
    plsc.subcore_barrier()


def _copy_out(acc_sh, out_hbm, cid, sid):
    plsc.subcore_barrier()
    pltpu.sync_copy(
        acc_sh.at[pl.ds(sid * RPT, RPT)],
        out_hbm.at[cid, pl.ds(sid * RPT, RPT)],
    )


@functools.partial(
    pl.kernel,
    mesh=_MESH,
    out_type=jax.ShapeDtypeStruct((NC, N_PAD, D_HID), jnp.float32),
    scratch_types=[
        pltpu.VMEM((PH, 2, C), jnp.int32),
        pltpu.VMEM((C, D_HID), jnp.float32),
        pltpu.VMEM_SHARED((N_PAD, D_HID), jnp.float32),
        pltpu.SemaphoreType.DMA,
        pltpu.SemaphoreType.DMA,
    ],
)
def _deg_sc(ei_hbm, out_hbm, idx_v, ones_v, acc_sh, sema, semb):
    cid = lax.axis_index("c")
    sid = lax.axis_index("s")
    wid = sid * NC + cid
    _zero_acc(ones_v, acc_sh, sid)
    _fill(ones_v, C, 1.0)

    def scat(ch, sem):
        pltpu.async_copy(ones_v, acc_sh.at[idx_v.at[ch, 1]], sem, add=True)

    def drain(sem):
        pltpu.make_async_copy(ones_v, acc_sh.at[idx_v.at[0, 1]], sem).wait()

    for off in range(0, NCH, PH):
        n = min(PH, NCH - off)
        pltpu.sync_copy(ei_hbm.at[wid, pl.ds(off, n)],
                        idx_v.at[pl.ds(0, n)])

        def pair(j, _):
            scat(2 * j, sema)
            scat(2 * j + 1, semb)
            drain(sema)
            drain(semb)
            return 0

        lax.fori_loop(0, n // 2, pair, 0)
        if n % 2:
            scat(n - 1, sema)
            drain(sema)
    _copy_out(acc_sh, out_hbm, cid, sid)


@functools.partial(
    pl.kernel,
    mesh=_MESH,
    out_type=jax.ShapeDtypeStruct((NC, N_PAD, D_HID), jnp.float32),
    scratch_types=[
        pltpu.VMEM((PH, 2, C), jnp.int32),
        pltpu.VMEM((C, D_HID), jnp.float32),
        pltpu.VMEM((C, D_HID), jnp.float32),
        pltpu.VMEM_SHARED((N_PAD, D_HID), jnp.float32),
        pltpu.SemaphoreType.DMA,
        pltpu.SemaphoreType.DMA,
    ],
)
def _scatter_sc(g_hbm, eif_hbm, eis_hbm, out_hbm, idx_v, rows_a, rows_b,
                acc_sh, sema, semb):
    cid = lax.axis_index("c")
    sid = lax.axis_index("s")
    _zero_acc(rows_a, acc_sh, sid)

    def gat(ch, rows, sem):
        pltpu.async_copy(g_hbm.at[idx_v.at[ch, 0]], rows, sem)

    def wt(rows, sem):
        pltpu.make_async_copy(g_hbm.at[idx_v.at[0, 0]], rows, sem).wait()

    def scat(ch, rows):
        pltpu.sync_copy(rows, acc_sh.at[idx_v.at[ch, 1]], add=True)

    def run(ei_hbm, total):
        for off in range(0, total, PH):
            n = min(PH, total - off)
            pltpu.sync_copy(ei_hbm.at[sid, pl.ds(off, n)],
                            idx_v.at[pl.ds(0, n)])
            gat(0, rows_a, sema)

            def pair(j, _):
                a = 2 * j
                wt(rows_a, sema)
                gat(a + 1, rows_b, semb)
                scat(a, rows_a)
                wt(rows_b, semb)
                gat(a + 2, rows_a, sema)
                scat(a + 1, rows_b)
                return 0

            npair = (n - 1) // 2
            lax.fori_loop(0, npair, pair, 0)
            if n % 2:
                wt(rows_a, sema)
                scat(n - 1, rows_a)
            else:
                wt(rows_a, sema)
                gat(n - 1, rows_b, semb)
                scat(n - 2, rows_a)
                wt(rows_b, semb)
                scat(n - 1, rows_b)

    @pl.when(cid == 0)
    def _():
        run(eif_hbm, CH_F)

    @pl.when(cid == 1)
    def _():
        run(eis_hbm, CH_S)

    _copy_out(acc_sh, out_hbm, cid, sid)


_BR = 1024
_G = N_PAD // _BR


def _rs(deg0, deg1):
    return lax.rsqrt(deg0[:, 0:1] + deg1[:, 0:1] + 1.0)


def _z1_body(deg0_ref, deg1_ref, x_ref, w_ref, z_ref):
    rs = _rs(deg0_ref[...], deg1_ref[...])
    z_ref[...] = jnp.dot(x_ref[...], w_ref[...],
                         preferred_element_type=jnp.float32) * rs


def _mid_body(deg0_ref, deg1_ref, s0_ref, s1_ref, z_ref, b_ref, w_ref, o_ref):
    rs = _rs(deg0_ref[...], deg1_ref[...])
    h = jax.nn.relu(rs * (s0_ref[...] + s1_ref[...] + z_ref[...]) + b_ref[...])
    o_ref[...] = jnp.dot(h, w_ref[...], preferred_element_type=jnp.float32) * rs


def _fin_body(deg0_ref, deg1_ref, s0_ref, s1_ref, z_ref, b_ref, w_ref, bf_ref,
              o_ref):
    rs = _rs(deg0_ref[...], deg1_ref[...])
    h = jax.nn.relu(rs * (s0_ref[...] + s1_ref[...] + z_ref[...]) + b_ref[...])
    o_ref[...] = jnp.dot(h, w_ref[...],
                         preferred_element_type=jnp.float32) + bf_ref[...]


def _row_spec(w):
    return pl.BlockSpec((_BR, w), lambda i: (i, 0))


def _full_spec(r, c):
    return pl.BlockSpec((r, c), lambda i: (0, 0))


_z1_call = pl.pallas_call(
    _z1_body,
    grid=(_G,),
    in_specs=[_row_spec(D_HID), _row_spec(D_HID), _row_spec(D_IN),
              _full_spec(D_IN, D_HID)],
    out_specs=_row_spec(D_HID),
    out_shape=jax.ShapeDtypeStruct((N_PAD, D_HID), jnp.float32),
)

_mid_call = pl.pallas_call(
    _mid_body,
    grid=(_G,),
    in_specs=[_row_spec(D_HID), _row_spec(D_HID), _row_spec(D_HID),
              _row_spec(D_HID), _row_spec(D_HID), _full_spec(1, D_HID),
              _full_spec(D_HID, D_HID)],
    out_specs=_row_spec(D_HID),
    out_shape=jax.ShapeDtypeStruct((N_PAD, D_HID), jnp.float32),
)

_fin_call = pl.pallas_call(
    _fin_body,
    grid=(_G,),
    in_specs=[_row_spec(D_HID), _row_spec(D_HID), _row_spec(D_HID),
              _row_spec(D_HID), _row_spec(D_HID), _full_spec(1, D_HID),
              _full_spec(D_HID, D_OUT), _full_spec(1, D_OUT)],
    out_specs=_row_spec(D_OUT),
    out_shape=jax.ShapeDtypeStruct((N_PAD, D_OUT), jnp.float32),
)


def kernel(x, edge_index, W1, b1, W2, b2, Wf, bf):
    src = edge_index[0].astype(jnp.int32)
    dst = edge_index[1].astype(jnp.int32)
    fill_src = (jnp.arange(E_PAD - E, dtype=jnp.int32) * 97) % N
    src_p = jnp.concatenate([src, fill_src]).reshape(CH_TOT, 1, C)
    junk = N + jnp.arange(E_PAD - E, dtype=jnp.int32) % (N_PAD - N)
    dst_p = jnp.concatenate([dst, junk]).reshape(CH_TOT, 1, C)
    ei = jnp.concatenate([src_p, dst_p], axis=2)
    ei4 = ei.reshape(NW, NCH, 2, C)
    ei_f = ei[:NS * CH_F].reshape(NS, CH_F, 2, C)
    ei_s = ei[NS * CH_F:].reshape(NS, CH_S, 2, C)
    x_p = jnp.pad(x, ((0, N_PAD - N), (0, 0)))

    deg_p = _deg_sc(ei4)
    deg0, deg1 = deg_p[0], deg_p[1]

    z1 = _z1_call(deg0, deg1, x_p, W1)
    s = _scatter_sc(z1, ei_f, ei_s)
    z2 = _mid_call(deg0, deg1, s[0], s[1], z1, b1.reshape(1, -1), W2)
    s2 = _scatter_sc(z2, ei_f, ei_s)
    out = _fin_call(deg0, deg1, s2[0], s2[1], z2, b2.reshape(1, -1), Wf,
                    bf.reshape(1, -1))
    return out[:N]


# speedup vs baseline: 1.6458x; 1.0008x over previous
import functools

import jax
import jax.numpy as jnp
from jax import lax
from jax.experimental import pallas as pl
from jax.experimental.pallas import tpu as pltpu
from jax.experimental.pallas import tpu_sc as plsc

N = 10000
E = 320000
D_IN = 128
D_HID = 128
D_OUT = 64

N_PAD = 10240
NC = 2
NS = 16
NW = NC * NS
C = 128
NCH = 79
CH_TOT = NCH * NW
E_PAD = CH_TOT * C
CH_F = 79
CH_S = CH_TOT // NS - CH_F
PH = 40
RPT = N_PAD // NS

_MESH = plsc.VectorSubcoreMesh(core_axis_name="c", subcore_axis_name="s")


def _fill(ref, rows, val):
    def body(k, _):
        ref[k // 8, pl.ds((k % 8) * 16, 16)] = jnp.full((16,), val, jnp.float32)
        return 0

    lax.fori_loop(0, rows * 8, body, 0)


def _zero_acc(zbuf_v, acc_sh, sid):
    _fill(zbuf_v, C, 0.0)

    def zcp(k, _):
        pltpu.sync_copy(zbuf_v, acc_sh.at[pl.ds(sid * RPT + k * C, C)])
        return 0

    lax.fori_loop(0, RPT // C, zcp, 0)
    plsc.subcore_barrier()


def _copy_out(acc_sh, out_hbm, cid, sid):
    plsc.subcore_barrier()
    pltpu.sync_copy(
        acc_sh.at[pl.ds(sid * RPT, RPT)],
        out_hbm.at[cid, pl.ds(sid * RPT, RPT)],
    )


@functools.partial(
    pl.kernel,
    mesh=_MESH,
    out_type=jax.ShapeDtypeStruct((NC, N_PAD, D_HID), jnp.float32),
    scratch_types=[
        pltpu.VMEM((PH, 2, C), jnp.int32),
        pltpu.VMEM((C, D_HID), jnp.float32),
        pltpu.VMEM_SHARED((N_PAD, D_HID), jnp.float32),
        pltpu.SemaphoreType.DMA,
        pltpu.SemaphoreType.DMA,
    ],
)
def _deg_sc(ei_hbm, out_hbm, idx_v, ones_v, acc_sh, sema, semb):
    cid = lax.axis_index("c")
    sid = lax.axis_index("s")
    wid = sid * NC + cid
    _zero_acc(ones_v, acc_sh, sid)
    _fill(ones_v, C, 1.0)

    def scat(ch, sem):
        pltpu.async_copy(ones_v, acc_sh.at[idx_v.at[ch, 1]], sem, add=True)

    def drain(sem):
        pltpu.make_async_copy(ones_v, acc_sh.at[idx_v.at[0, 1]], sem).wait()

    for off in range(0, NCH, PH):
        n = min(PH, NCH - off)
        pltpu.sync_copy(ei_hbm.at[wid, pl.ds(off, n)],
                        idx_v.at[pl.ds(0, n)])

        def pair(j, _):
            scat(2 * j, sema)
            scat(2 * j + 1, semb)
            drain(sema)
            drain(semb)
            return 0

        lax.fori_loop(0, n // 2, pair, 0)
        if n % 2:
            scat(n - 1, sema)
            drain(sema)
    _copy_out(acc_sh, out_hbm, cid, sid)


@functools.partial(
    pl.kernel,
    mesh=_MESH,
    out_type=jax.ShapeDtypeStruct((NC, N_PAD, D_HID), jnp.float32),
    scratch_types=[
        pltpu.VMEM((PH, 2, C), jnp.int32),
        pltpu.VMEM((C, D_HID), jnp.float32),
        pltpu.VMEM((C, D_HID), jnp.float32),
        pltpu.VMEM_SHARED((N_PAD, D_HID), jnp.float32),
        pltpu.SemaphoreType.DMA,
        pltpu.SemaphoreType.DMA,
        pltpu.SemaphoreType.DMA,
        pltpu.SemaphoreType.DMA,
    ],
)
def _scatter_sc(g_hbm, eif_hbm, eis_hbm, out_hbm, idx_v, rows_a, rows_b,
                acc_sh, sga, sgb, ssa, ssb):
    cid = lax.axis_index("c")
    sid = lax.axis_index("s")
    _zero_acc(rows_a, acc_sh, sid)

    def gat(ch, rows, sem):
        pltpu.async_copy(g_hbm.at[idx_v.at[ch, 0]], rows, sem)

    def wtg(rows, sem):
        pltpu.make_async_copy(g_hbm.at[idx_v.at[0, 0]], rows, sem).wait()

    def scat(ch, rows, sem):
        pltpu.async_copy(rows, acc_sh.at[idx_v.at[ch, 1]], sem, add=True)

    def wts(rows, sem):
        pltpu.make_async_copy(rows, acc_sh.at[idx_v.at[0, 1]], sem).wait()

    def run(ei_hbm, total):
        for off in range(0, total, PH):
            n = min(PH, total - off)
            pltpu.sync_copy(ei_hbm.at[sid, pl.ds(off, n)],
                            idx_v.at[pl.ds(0, n)])
            gat(0, rows_a, sga)
            wtg(rows_a, sga)
            scat(0, rows_a, ssa)
            if n == 1:
                wts(rows_a, ssa)
                continue
            gat(1, rows_b, sgb)
            nj = max(0, (n - 4) // 2 + 1)

            def pair(j, _):
                a = 2 * j + 1
                wtg(rows_b, sgb)
                scat(a, rows_b, ssb)
                wts(rows_a, ssa)
                gat(a + 1, rows_a, sga)
                wtg(rows_a, sga)
                scat(a + 1, rows_a, ssa)
                wts(rows_b, ssb)
                gat(a + 2, rows_b, sgb)
                return 0

            lax.fori_loop(0, nj, pair, 0)
            for ch in range(2 * nj + 1, n):
                buf, gsem, ssem = ((rows_a, sga, ssa) if ch % 2 == 0
                                   else (rows_b, sgb, ssb))
                obuf, ogsem, ossem = ((rows_a, sga, ssa) if ch % 2 == 1
                                      else (rows_b, sgb, ssb))
                wtg(buf, gsem)
                scat(ch, buf, ssem)
                if ch + 1 < n:
                    wts(obuf, ossem)
                    gat(ch + 1, obuf, ogsem)
            wts(rows_a, ssa)
            wts(rows_b, ssb)

    @pl.when(cid == 0)
    def _():
        run(eif_hbm, CH_F)

    @pl.when(cid == 1)
    def _():
        run(eis_hbm, CH_S)

    _copy_out(acc_sh, out_hbm, cid, sid)


_BR = 1024
_G = N_PAD // _BR


def _rs(deg0, deg1):
    return lax.rsqrt(deg0[:, 0:1] + deg1[:, 0:1] + 1.0)


def _z1_body(deg0_ref, deg1_ref, x_ref, w_ref, z_ref):
    rs = _rs(deg0_ref[...], deg1_ref[...])
    z_ref[...] = jnp.dot(x_ref[...], w_ref[...],
                         preferred_element_type=jnp.float32) * rs


def _mid_body(deg0_ref, deg1_ref, s0_ref, s1_ref, z_ref, b_ref, w_ref, o_ref):
    rs = _rs(deg0_ref[...], deg1_ref[...])
    h = jax.nn.relu(rs * (s0_ref[...] + s1_ref[...] + z_ref[...]) + b_ref[...])
    o_ref[...] = jnp.dot(h, w_ref[...], preferred_element_type=jnp.float32) * rs


def _fin_body(deg0_ref, deg1_ref, s0_ref, s1_ref, z_ref, b_ref, w_ref, bf_ref,
              o_ref):
    rs = _rs(deg0_ref[...], deg1_ref[...])
    h = jax.nn.relu(rs * (s0_ref[...] + s1_ref[...] + z_ref[...]) + b_ref[...])
    o_ref[...] = jnp.dot(h, w_ref[...],
                         preferred_element_type=jnp.float32) + bf_ref[...]


def _row_spec(w):
    return pl.BlockSpec((_BR, w), lambda i: (i, 0))


def _full_spec(r, c):
    return pl.BlockSpec((r, c), lambda i: (0, 0))


_z1_call = pl.pallas_call(
    _z1_body,
    grid=(_G,),
    in_specs=[_row_spec(D_HID), _row_spec(D_HID), _row_spec(D_IN),
              _full_spec(D_IN, D_HID)],
    out_specs=_row_spec(D_HID),
    out_shape=jax.ShapeDtypeStruct((N_PAD, D_HID), jnp.float32),
)

_mid_call = pl.pallas_call(
    _mid_body,
    grid=(_G,),
    in_specs=[_row_spec(D_HID), _row_spec(D_HID), _row_spec(D_HID),
              _row_spec(D_HID), _row_spec(D_HID), _full_spec(1, D_HID),
              _full_spec(D_HID, D_HID)],
    out_specs=_row_spec(D_HID),
    out_shape=jax.ShapeDtypeStruct((N_PAD, D_HID), jnp.float32),
)

_fin_call = pl.pallas_call(
    _fin_body,
    grid=(_G,),
    in_specs=[_row_spec(D_HID), _row_spec(D_HID), _row_spec(D_HID),
              _row_spec(D_HID), _row_spec(D_HID), _full_spec(1, D_HID),
              _full_spec(D_HID, D_OUT), _full_spec(1, D_OUT)],
    out_specs=_row_spec(D_OUT),
    out_shape=jax.ShapeDtypeStruct((N_PAD, D_OUT), jnp.float32),
)


def kernel(x, edge_index, W1, b1, W2, b2, Wf, bf):
    src = edge_index[0].astype(jnp.int32)
    dst = edge_index[1].astype(jnp.int32)
    fill_src = (jnp.arange(E_PAD - E, dtype=jnp.int32) * 97) % N
    src_p = jnp.concatenate([src, fill_src]).reshape(CH_TOT, 1, C)
    junk = N + jnp.arange(E_PAD - E, dtype=jnp.int32) % (N_PAD - N)
    dst_p = jnp.concatenate([dst, junk]).reshape(CH_TOT, 1, C)
    ei = jnp.concatenate([src_p, dst_p], axis=2)
    ei4 = ei.reshape(NW, NCH, 2, C)
    ei_f = ei[:NS * CH_F].reshape(NS, CH_F, 2, C)
    ei_s = ei[NS * CH_F:].reshape(NS, CH_S, 2, C)
    x_p = jnp.pad(x, ((0, N_PAD - N), (0, 0)))

    deg_p = _deg_sc(ei4)
    deg0, deg1 = deg_p[0], deg_p[1]

    z1 = _z1_call(deg0, deg1, x_p, W1)
    s = _scatter_sc(z1, ei_f, ei_s)
    z2 = _mid_call(deg0, deg1, s[0], s[1], z1, b1.reshape(1, -1), W2)
    s2 = _scatter_sc(z2, ei_f, ei_s)
    out = _fin_call(deg0, deg1, s2[0], s2[1], z2, b2.reshape(1, -1), Wf,
                    bf.reshape(1, -1))
    return out[:N]


# speedup vs baseline: 1.6907x; 1.0273x over previous
import functools

import jax
import jax.numpy as jnp
from jax import lax
from jax.experimental import pallas as pl
from jax.experimental.pallas import tpu as pltpu
from jax.experimental.pallas import tpu_sc as plsc

N = 10000
E = 320000
D_IN = 128
D_HID = 128
D_OUT = 64

N_PAD = 10240
NC = 2
NS = 16
NW = NC * NS
C = 128
NCH = 79
CH_TOT = NCH * NW
E_PAD = CH_TOT * C
CH_F = 79
CH_S = CH_TOT // NS - CH_F
PH = 40
RPT = N_PAD // NS

_MESH = plsc.VectorSubcoreMesh(core_axis_name="c", subcore_axis_name="s")


def _fill(ref, rows, val):
    def body(k, _):
        for j in range(8):
            ref[k, pl.ds(j * 16, 16)] = jnp.full((16,), val, jnp.float32)
        return 0

    lax.fori_loop(0, rows, body, 0)


def _zero_acc(zbuf_v, acc_sh, sid):
    _fill(zbuf_v, C, 0.0)

    def zcp(k, _):
        pltpu.sync_copy(zbuf_v, acc_sh.at[pl.ds(sid * RPT + k * C, C)])
        return 0

    lax.fori_loop(0, RPT // C, zcp, 0)
    plsc.subcore_barrier()


def _copy_out(acc_sh, out_hbm, cid, sid):
    plsc.subcore_barrier()
    pltpu.sync_copy(
        acc_sh.at[pl.ds(sid * RPT, RPT)],
        out_hbm.at[cid, pl.ds(sid * RPT, RPT)],
    )


@functools.partial(
    pl.kernel,
    mesh=_MESH,
    out_type=jax.ShapeDtypeStruct((NC, N_PAD, D_HID), jnp.float32),
    scratch_types=[
        pltpu.VMEM((PH, 2, C), jnp.int32),
        pltpu.VMEM((C, D_HID), jnp.float32),
        pltpu.VMEM_SHARED((N_PAD, D_HID), jnp.float32),
        pltpu.SemaphoreType.DMA,
        pltpu.SemaphoreType.DMA,
    ],
)
def _deg_sc(ei_hbm, out_hbm, idx_v, ones_v, acc_sh, sema, semb):
    cid = lax.axis_index("c")
    sid = lax.axis_index("s")
    wid = sid * NC + cid
    _zero_acc(ones_v, acc_sh, sid)
    _fill(ones_v, C, 1.0)

    def scat(ch, sem):
        pltpu.async_copy(ones_v, acc_sh.at[idx_v.at[ch, 1]], sem, add=True)

    def drain(sem):
        pltpu.make_async_copy(ones_v, acc_sh.at[idx_v.at[0, 1]], sem).wait()

    for off in range(0, NCH, PH):
        n = min(PH, NCH - off)
        pltpu.sync_copy(ei_hbm.at[wid, pl.ds(off, n)],
                        idx_v.at[pl.ds(0, n)])

        def pair(j, _):
            scat(2 * j, sema)
            scat(2 * j + 1, semb)
            drain(sema)
            drain(semb)
            return 0

        lax.fori_loop(0, n // 2, pair, 0)
        if n % 2:
            scat(n - 1, sema)
            drain(sema)
    _copy_out(acc_sh, out_hbm, cid, sid)


@functools.partial(
    pl.kernel,
    mesh=_MESH,
    out_type=jax.ShapeDtypeStruct((NC, N_PAD, D_HID), jnp.float32),
    scratch_types=[
        pltpu.VMEM((PH, 2, C), jnp.int32),
        pltpu.VMEM((C, D_HID), jnp.float32),
        pltpu.VMEM((C, D_HID), jnp.float32),
        pltpu.VMEM_SHARED((N_PAD, D_HID), jnp.float32),
        pltpu.SemaphoreType.DMA,
        pltpu.SemaphoreType.DMA,
        pltpu.SemaphoreType.DMA,
        pltpu.SemaphoreType.DMA,
    ],
)
def _scatter_sc(g_hbm, eif_hbm, eis_hbm, out_hbm, idx_v, rows_a, rows_b,
                acc_sh, sga, sgb, ssa, ssb):
    cid = lax.axis_index("c")
    sid = lax.axis_index("s")
    _zero_acc(rows_a, acc_sh, sid)

    def gat(ch, rows, sem):
        pltpu.async_copy(g_hbm.at[idx_v.at[ch, 0]], rows, sem)

    def wtg(rows, sem):
        pltpu.make_async_copy(g_hbm.at[idx_v.at[0, 0]], rows, sem).wait()

    def scat(ch, rows, sem):
        pltpu.async_copy(rows, acc_sh.at[idx_v.at[ch, 1]], sem, add=True)

    def wts(rows, sem):
        pltpu.make_async_copy(rows, acc_sh.at[idx_v.at[0, 1]], sem).wait()

    def run(ei_hbm, total):
        for off in range(0, total, PH):
            n = min(PH, total - off)
            pltpu.sync_copy(ei_hbm.at[sid, pl.ds(off, n)],
                            idx_v.at[pl.ds(0, n)])
            gat(0, rows_a, sga)
            wtg(rows_a, sga)
            scat(0, rows_a, ssa)
            if n == 1:
                wts(rows_a, ssa)
                continue
            gat(1, rows_b, sgb)
            nj = max(0, (n - 4) // 2 + 1)

            def pair(j, _):
                a = 2 * j + 1
                wtg(rows_b, sgb)
                scat(a, rows_b, ssb)
                wts(rows_a, ssa)
                gat(a + 1, rows_a, sga)
                wtg(rows_a, sga)
                scat(a + 1, rows_a, ssa)
                wts(rows_b, ssb)
                gat(a + 2, rows_b, sgb)
                return 0

            lax.fori_loop(0, nj, pair, 0)
            for ch in range(2 * nj + 1, n):
                buf, gsem, ssem = ((rows_a, sga, ssa) if ch % 2 == 0
                                   else (rows_b, sgb, ssb))
                obuf, ogsem, ossem = ((rows_a, sga, ssa) if ch % 2 == 1
                                      else (rows_b, sgb, ssb))
                wtg(buf, gsem)
                scat(ch, buf, ssem)
                if ch + 1 < n:
                    wts(obuf, ossem)
                    gat(ch + 1, obuf, ogsem)
            wts(rows_a, ssa)
            wts(rows_b, ssb)

    @pl.when(cid == 0)
    def _():
        run(eif_hbm, CH_F)

    @pl.when(cid == 1)
    def _():
        run(eis_hbm, CH_S)

    _copy_out(acc_sh, out_hbm, cid, sid)


_BR = 1024
_G = N_PAD // _BR


def _rs(deg0, deg1):
    return lax.rsqrt(deg0[:, 0:1] + deg1[:, 0:1] + 1.0)


def _z1_body(deg0_ref, deg1_ref, x_ref, w_ref, z_ref, rs8_ref):
    rs = _rs(deg0_ref[...], deg1_ref[...])
    z_ref[...] = jnp.dot(x_ref[...], w_ref[...],
                         preferred_element_type=jnp.float32) * rs
    rs8_ref[...] = jnp.broadcast_to(rs, (_BR, 8))


def _mid_body(rs8_ref, s0_ref, s1_ref, z_ref, b_ref, w_ref, o_ref):
    rs = rs8_ref[:, 0:1]
    h = jax.nn.relu(rs * (s0_ref[...] + s1_ref[...] + z_ref[...]) + b_ref[...])
    o_ref[...] = jnp.dot(h, w_ref[...], preferred_element_type=jnp.float32) * rs


def _fin_body(rs8_ref, s0_ref, s1_ref, z_ref, b_ref, w_ref, bf_ref,
              o_ref):
    rs = rs8_ref[:, 0:1]
    h = jax.nn.relu(rs * (s0_ref[...] + s1_ref[...] + z_ref[...]) + b_ref[...])
    o_ref[...] = jnp.dot(h, w_ref[...],
                         preferred_element_type=jnp.float32) + bf_ref[...]


def _row_spec(w):
    return pl.BlockSpec((_BR, w), lambda i: (i, 0))


def _full_spec(r, c):
    return pl.BlockSpec((r, c), lambda i: (0, 0))


_z1_call = pl.pallas_call(
    _z1_body,
    grid=(_G,),
    in_specs=[_row_spec(D_HID), _row_spec(D_HID), _row_spec(D_IN),
              _full_spec(D_IN, D_HID)],
    out_specs=[_row_spec(D_HID), _row_spec(8)],
    out_shape=[jax.ShapeDtypeStruct((N_PAD, D_HID), jnp.float32),
               jax.ShapeDtypeStruct((N_PAD, 8), jnp.float32)],
)

_mid_call = pl.pallas_call(
    _mid_body,
    grid=(_G,),
    in_specs=[_row_spec(8), _row_spec(D_HID),
              _row_spec(D_HID), _row_spec(D_HID), _full_spec(1, D_HID),
              _full_spec(D_HID, D_HID)],
    out_specs=_row_spec(D_HID),
    out_shape=jax.ShapeDtypeStruct((N_PAD, D_HID), jnp.float32),
)

_fin_call = pl.pallas_call(
    _fin_body,
    grid=(_G,),
    in_specs=[_row_spec(8), _row_spec(D_HID),
              _row_spec(D_HID), _row_spec(D_HID), _full_spec(1, D_HID),
              _full_spec(D_HID, D_OUT), _full_spec(1, D_OUT)],
    out_specs=_row_spec(D_OUT),
    out_shape=jax.ShapeDtypeStruct((N_PAD, D_OUT), jnp.float32),
)


def kernel(x, edge_index, W1, b1, W2, b2, Wf, bf):
    src = edge_index[0].astype(jnp.int32)
    dst = edge_index[1].astype(jnp.int32)
    fill_src = (jnp.arange(E_PAD - E, dtype=jnp.int32) * 97) % N
    src_p = jnp.concatenate([src, fill_src]).reshape(CH_TOT, 1, C)
    junk = N + jnp.arange(E_PAD - E, dtype=jnp.int32) % (N_PAD - N)
    dst_p = jnp.concatenate([dst, junk]).reshape(CH_TOT, 1, C)
    ei = jnp.concatenate([src_p, dst_p], axis=2)
    ei4 = ei.reshape(NW, NCH, 2, C)
    ei_f = ei[:NS * CH_F].reshape(NS, CH_F, 2, C)
    ei_s = ei[NS * CH_F:].reshape(NS, CH_S, 2, C)
    x_p = jnp.pad(x, ((0, N_PAD - N), (0, 0)))

    deg_p = _deg_sc(ei4)
    deg0, deg1 = deg_p[0], deg_p[1]

    z1, rs8 = _z1_call(deg0, deg1, x_p, W1)
    s = _scatter_sc(z1, ei_f, ei_s)
    z2 = _mid_call(rs8, s[0], s[1], z1, b1.reshape(1, -1), W2)
    s2 = _scatter_sc(z2, ei_f, ei_s)
    out = _fin_call(rs8, s2[0], s2[1], z2, b2.reshape(1, -1), Wf,
                    bf.reshape(1, -1))
    return out[:N]
